# baseline (speedup 1.0000x reference)
import functools

import jax
import jax.numpy as jnp
from jax import lax
from jax.experimental import pallas as pl
from jax.experimental.pallas import tpu as pltpu
from jax.experimental.pallas import tpu_sc as plsc

D = 128
LANES = 16
COLB = D // LANES


def _make_sc_kernel(b, t, dp, vocab):
    info = plsc.get_sparse_core_info()
    nw = info.num_cores * info.num_subcores
    bpw = b // nw
    k = t * dp
    kpad = (k + 2 * LANES - 1) // LANES * LANES
    ngrp = kpad // LANES

    mesh = plsc.VectorSubcoreMesh(core_axis_name="c", subcore_axis_name="s")

    @functools.partial(
        pl.kernel,
        mesh=mesh,
        out_type=jax.ShapeDtypeStruct((b * t, dp, D), jnp.float32),
        compiler_params=pltpu.CompilerParams(
            use_tc_tiling_on_sc=True, needs_layout_passes=False),
        scratch_types=[
            pltpu.VMEM((bpw * k,), jnp.int32),
            pltpu.VMEM((bpw * k,), jnp.float32),
            pltpu.VMEM((kpad,), jnp.int32),
            pltpu.VMEM((kpad, D), jnp.float32),
            pltpu.VMEM((k, D), jnp.float32),
            pltpu.SemaphoreType.DMA,
            pltpu.SemaphoreType.DMA,
        ],
    )
    def sc_embed(table_hbm, idx_hbm, num_hbm, out_hbm,
                 idx_all, num_all, idx_st, g_v, o_v, sem, wsem):
        wid = lax.axis_index("s") * info.num_cores + lax.axis_index("c")
        wbase = wid * (bpw * k)
        pltpu.sync_copy(idx_hbm.at[pl.ds(wbase, bpw * k)], idx_all)
        pltpu.sync_copy(num_hbm.at[pl.ds(wbase, bpw * k)], num_all)
        lanes = lax.iota(jnp.int32, LANES)

        def b_body(i, _):
            cbase = i * k
            for g in range(ngrp):
                v = plsc.load_gather(idx_all, [jnp.minimum(lanes + (cbase + g * LANES), bpw * k - 1)])
                idx_st[pl.ds(g * LANES, LANES)] = jnp.clip(v, 0, vocab - 1)
            pltpu.async_copy(table_hbm.at[idx_st], g_v, sem).wait()

            bb = wid * bpw + i

            def t_body(tt, _):
                for d in range(dp):
                    row = tt * dp + d
                    s16 = plsc.load_gather(num_all, [jnp.full((LANES,), cbase + row, jnp.int32)])
                    for kk in range(COLB):
                        blk = g_v[row, pl.ds(kk * LANES, LANES)]
                        o_v[row, pl.ds(kk * LANES, LANES)] = blk * s16
                pltpu.async_copy(o_v.at[pl.ds(tt * dp, dp)], out_hbm.at[bb * t + tt], wsem)
                return 0

            lax.fori_loop(0, t, t_body, 0)

            def drain_body(tt, _):
                pltpu.make_async_copy(o_v.at[pl.ds(tt * dp, dp)], out_hbm.at[bb * t + tt], wsem).wait()
                return 0

            lax.fori_loop(0, t, drain_body, 0)
            return 0

        lax.fori_loop(0, bpw, b_body, 0)

    return sc_embed


def kernel(idx, num, table):
    b, t, dp = idx.shape
    n = b * t * dp
    vocab = table.shape[0]
    idx_flat = idx.reshape(n).astype(jnp.int32)
    num_flat = num.reshape(n)
    out = _make_sc_kernel(b, t, dp, vocab)(table, idx_flat, num_flat)
    return out.reshape(b, t, dp, D)


# speedup vs baseline: 1.2257x; 1.2257x over previous
import functools

import jax
import jax.numpy as jnp
from jax import lax
from jax.experimental import pallas as pl
from jax.experimental.pallas import tpu as pltpu
from jax.experimental.pallas import tpu_sc as plsc

D = 128
LANES = 16
COLB = D // LANES


def _make_sc_kernel(b, t, dp):
    info = plsc.get_sparse_core_info()
    nw = info.num_cores * info.num_subcores
    bpw = b // nw
    k = t * dp
    kpad = (k + LANES - 1) // LANES * LANES
    ngrp = kpad // LANES
    npw = bpw * k
    assert bpw % 2 == 0

    mesh = plsc.VectorSubcoreMesh(core_axis_name="c", subcore_axis_name="s")

    @functools.partial(
        pl.kernel,
        mesh=mesh,
        out_type=jax.ShapeDtypeStruct((b * t, dp, D), jnp.float32),
        compiler_params=pltpu.CompilerParams(needs_layout_passes=False),
        scratch_types=[
            pltpu.VMEM((npw,), jnp.int32),
            pltpu.VMEM((npw,), jnp.float32),
            pltpu.VMEM((kpad,), jnp.int32),
            pltpu.VMEM((kpad,), jnp.int32),
            pltpu.VMEM((kpad, D), jnp.float32),
            pltpu.VMEM((kpad, D), jnp.float32),
            pltpu.SemaphoreType.DMA,
            pltpu.SemaphoreType.DMA,
            pltpu.SemaphoreType.DMA,
            pltpu.SemaphoreType.DMA,
        ],
    )
    def sc_embed(table_hbm, idx_hbm, num_hbm, out_hbm,
                 idx_all, num_all, st0, st1, g0, g1, gs0, gs1, ws0, ws1):
        st = (st0, st1)
        g = (g0, g1)
        gsem = (gs0, gs1)
        wsem = (ws0, ws1)
        wid = lax.axis_index("s") * info.num_cores + lax.axis_index("c")
        wbase = wid * npw
        b0 = wid * bpw
        pltpu.sync_copy(idx_hbm.at[pl.ds(wbase, npw)], idx_all)
        pltpu.sync_copy(num_hbm.at[pl.ds(wbase, npw)], num_all)
        lanes = lax.iota(jnp.int32, LANES)

        def stage(i, sp):
            cbase = i * k
            for gg in range(ngrp):
                pos = jnp.minimum(lanes + (cbase + gg * LANES), npw - 1)
                sp[pl.ds(gg * LANES, LANES)] = plsc.load_gather(idx_all, [pos])

        stage(0, st[0])
        pltpu.async_copy(table_hbm.at[st[0]], g[0], gsem[0])

        def pair_body(s, _):
            for p in range(2):
                i = 2 * s + p
                q = 1 - p

                def drain_body(tt, _):
                    pltpu.make_async_copy(
                        g[q].at[pl.ds(tt * dp, dp)],
                        out_hbm.at[(b0 + i - 1) * t + tt], wsem[q]).wait()
                    return 0

                if p == 0:
                    @pl.when(s >= 1)
                    def _():
                        lax.fori_loop(0, t, drain_body, 0)
                else:
                    lax.fori_loop(0, t, drain_body, 0)

                def start_next():
                    stage(i + 1, st[q])
                    pltpu.async_copy(table_hbm.at[st[q]], g[q], gsem[q])

                if p == 0:
                    start_next()
                else:
                    @pl.when(s < bpw // 2 - 1)
                    def _():
                        start_next()

                pltpu.make_async_copy(
                    table_hbm.at[st[p]], g[p], gsem[p]).wait()

                def t_body(tt, _):
                    for d in range(dp):
                        row = tt * dp + d
                        s16 = plsc.load_gather(
                            num_all, [jnp.full((LANES,), i * k + row, jnp.int32)])
                        for kk in range(COLB):
                            blk = g[p][row, pl.ds(kk * LANES, LANES)]
                            g[p][row, pl.ds(kk * LANES, LANES)] = blk * s16
                    pltpu.async_copy(
                        g[p].at[pl.ds(tt * dp, dp)],
                        out_hbm.at[(b0 + i) * t + tt], wsem[p])
                    return 0

                lax.fori_loop(0, t, t_body, 0)
            return 0

        lax.fori_loop(0, bpw // 2, pair_body, 0)

        def drain_last(tt, _):
            pltpu.make_async_copy(
                g[1].at[pl.ds(tt * dp, dp)],
                out_hbm.at[(b0 + bpw - 1) * t + tt], wsem[1]).wait()
            return 0

        lax.fori_loop(0, t, drain_last, 0)

    return sc_embed


def kernel(idx, num, table):
    b, t, dp = idx.shape
    n = b * t * dp
    idx_flat = idx.reshape(n).astype(jnp.int32)
    num_flat = num.reshape(n)
    out = _make_sc_kernel(b, t, dp)(table, idx_flat, num_flat)
    return out.reshape(b, t, dp, D)


# speedup vs baseline: 1.7581x; 1.4344x over previous
import functools

import jax
import jax.numpy as jnp
from jax import lax
from jax.experimental import pallas as pl
from jax.experimental.pallas import tpu as pltpu
from jax.experimental.pallas import tpu_sc as plsc

D = 128
LANES = 16
COLB = D // LANES


def _make_sc_kernel(b, t, dp):
    info = plsc.get_sparse_core_info()
    nw = info.num_cores * info.num_subcores
    bpw = b // nw
    k = t * dp
    nfull = k // LANES
    ktail = k - nfull * LANES
    npw = bpw * k
    assert bpw % 2 == 0

    mesh = plsc.VectorSubcoreMesh(core_axis_name="c", subcore_axis_name="s")

    @functools.partial(
        pl.kernel,
        mesh=mesh,
        out_type=jax.ShapeDtypeStruct((b * t, dp, D), jnp.float32),
        compiler_params=pltpu.CompilerParams(needs_layout_passes=False),
        scratch_types=[
            pltpu.VMEM((npw,), jnp.int32),
            pltpu.VMEM((npw,), jnp.float32),
            pltpu.VMEM((k,), jnp.int32),
            pltpu.VMEM((k,), jnp.int32),
            pltpu.VMEM((k, D), jnp.float32),
            pltpu.VMEM((k, D), jnp.float32),
            pltpu.SemaphoreType.DMA,
            pltpu.SemaphoreType.DMA,
            pltpu.SemaphoreType.DMA,
            pltpu.SemaphoreType.DMA,
        ],
    )
    def sc_embed(table_hbm, idx_hbm, num_hbm, out_hbm,
                 idx_all, num_all, st0, st1, g0, g1, gs0, gs1, ws0, ws1):
        st = (st0, st1)
        g = (g0, g1)
        gsem = (gs0, gs1)
        wsem = (ws0, ws1)
        wid = lax.axis_index("s") * info.num_cores + lax.axis_index("c")
        wbase = wid * npw
        b0 = wid * bpw
        pltpu.sync_copy(idx_hbm.at[pl.ds(wbase, npw)], idx_all)
        pltpu.sync_copy(num_hbm.at[pl.ds(wbase, npw)], num_all)
        lanes = lax.iota(jnp.int32, LANES)

        def stage(i, sp):
            cbase = i * k
            for gg in range(nfull):
                pos = lanes + (cbase + gg * LANES)
                sp[pl.ds(gg * LANES, LANES)] = plsc.load_gather(idx_all, [pos])
            pos = jnp.minimum(lanes + (cbase + nfull * LANES), npw - 1)
            v = plsc.load_gather(idx_all, [pos])
            plsc.store_scatter(sp, [lanes + nfull * LANES], v, mask=lanes < ktail)

        stage(0, st[0])
        pltpu.async_copy(table_hbm.at[st[0]], g[0], gsem[0])

        def pair_body(s, _):
            for p in range(2):
                i = 2 * s + p
                q = 1 - p

                def drain_prev():
                    pltpu.make_async_copy(
                        g[q].reshape(t, dp, D),
                        out_hbm.at[pl.ds((b0 + i - 1) * t, t)], wsem[q]).wait()

                if p == 0:
                    @pl.when(s >= 1)
                    def _():
                        drain_prev()
                else:
                    drain_prev()

                def start_next():
                    stage(i + 1, st[q])
                    pltpu.async_copy(table_hbm.at[st[q]], g[q], gsem[q])

                if p == 0:
                    start_next()
                else:
                    @pl.when(s < bpw // 2 - 1)
                    def _():
                        start_next()

                pltpu.make_async_copy(
                    table_hbm.at[st[p]], g[p], gsem[p]).wait()

                def t_body(tt, _):
                    for d in range(dp):
                        row = tt * dp + d
                        s16 = plsc.load_gather(
                            num_all, [jnp.full((LANES,), i * k + row, jnp.int32)])
                        for kk in range(COLB):
                            blk = g[p][row, pl.ds(kk * LANES, LANES)]
                            g[p][row, pl.ds(kk * LANES, LANES)] = blk * s16
                    return 0

                lax.fori_loop(0, t, t_body, 0)
                pltpu.async_copy(
                    g[p].reshape(t, dp, D),
                    out_hbm.at[pl.ds((b0 + i) * t, t)], wsem[p])
            return 0

        lax.fori_loop(0, bpw // 2, pair_body, 0)

        pltpu.make_async_copy(
            g[1].reshape(t, dp, D),
            out_hbm.at[pl.ds((b0 + bpw - 1) * t, t)], wsem[1]).wait()

    return sc_embed


def kernel(idx, num, table):
    b, t, dp = idx.shape
    n = b * t * dp
    idx_flat = idx.reshape(n).astype(jnp.int32)
    num_flat = num.reshape(n)
    out = _make_sc_kernel(b, t, dp)(table, idx_flat, num_flat)
    return out.reshape(b, t, dp, D)


# speedup vs baseline: 1.8842x; 1.0717x over previous
import functools

import jax
import jax.numpy as jnp
from jax import lax
from jax.experimental import pallas as pl
from jax.experimental.pallas import tpu as pltpu
from jax.experimental.pallas import tpu_sc as plsc

D = 128
LANES = 16
COLB = D // LANES


def _make_sc_kernel(b, t, dp, vocab):
    info = plsc.get_sparse_core_info()
    nw = info.num_cores * info.num_subcores
    bpw = b // nw
    k = t * dp
    nfull = k // LANES
    ktail = k - nfull * LANES
    npw = bpw * k
    assert bpw % 2 == 0

    mesh = plsc.VectorSubcoreMesh(core_axis_name="c", subcore_axis_name="s")

    @functools.partial(
        pl.kernel,
        mesh=mesh,
        out_type=jax.ShapeDtypeStruct((b * t, dp, D), jnp.float32),
        compiler_params=pltpu.CompilerParams(needs_layout_passes=False),
        scratch_types=[
            pltpu.VMEM((npw,), jnp.int32),
            pltpu.VMEM((npw,), jnp.float32),
            pltpu.VMEM((k,), jnp.int32),
            pltpu.VMEM((k,), jnp.int32),
            pltpu.VMEM((k, D), jnp.float32),
            pltpu.VMEM((k, D), jnp.float32),
            pltpu.VMEM_SHARED((vocab, D), jnp.float32),
            pltpu.SemaphoreType.DMA,
            pltpu.SemaphoreType.DMA,
            pltpu.SemaphoreType.DMA,
            pltpu.SemaphoreType.DMA,
        ],
    )
    def sc_embed(table_hbm, idx_hbm, num_hbm, out_hbm,
                 idx_all, num_all, st0, st1, g0, g1, table_sp,
                 gs0, gs1, ws0, ws1):
        st = (st0, st1)
        g = (g0, g1)
        gsem = (gs0, gs1)
        wsem = (ws0, ws1)
        wid = lax.axis_index("s") * info.num_cores + lax.axis_index("c")
        wbase = wid * npw
        b0 = wid * bpw
        sid = lax.axis_index("s")
        vpt = vocab // info.num_subcores
        pltpu.sync_copy(table_hbm.at[pl.ds(sid * vpt, vpt)],
                        table_sp.at[pl.ds(sid * vpt, vpt)])
        pltpu.sync_copy(idx_hbm.at[pl.ds(wbase, npw)], idx_all)
        pltpu.sync_copy(num_hbm.at[pl.ds(wbase, npw)], num_all)
        plsc.subcore_barrier()
        lanes = lax.iota(jnp.int32, LANES)

        def stage(i, sp):
            cbase = i * k
            for gg in range(nfull):
                pos = lanes + (cbase + gg * LANES)
                sp[pl.ds(gg * LANES, LANES)] = plsc.load_gather(idx_all, [pos])
            pos = jnp.minimum(lanes + (cbase + nfull * LANES), npw - 1)
            v = plsc.load_gather(idx_all, [pos])
            plsc.store_scatter(sp, [lanes + nfull * LANES], v, mask=lanes < ktail)

        stage(0, st[0])
        pltpu.async_copy(table_sp.at[st[0]], g[0], gsem[0])

        def pair_body(s, _):
            for p in range(2):
                i = 2 * s + p
                q = 1 - p

                def drain_prev():
                    pltpu.make_async_copy(
                        g[q].reshape(t, dp, D),
                        out_hbm.at[pl.ds((b0 + i - 1) * t, t)], wsem[q]).wait()

                if p == 0:
                    @pl.when(s >= 1)
                    def _():
                        drain_prev()
                else:
                    drain_prev()

                def start_next():
                    stage(i + 1, st[q])
                    pltpu.async_copy(table_sp.at[st[q]], g[q], gsem[q])

                if p == 0:
                    start_next()
                else:
                    @pl.when(s < bpw // 2 - 1)
                    def _():
                        start_next()

                pltpu.make_async_copy(
                    table_sp.at[st[p]], g[p], gsem[p]).wait()

                def t_body(tt, _):
                    for d in range(dp):
                        row = tt * dp + d
                        s16 = plsc.load_gather(
                            num_all, [jnp.full((LANES,), i * k + row, jnp.int32)])
                        for kk in range(COLB):
                            blk = g[p][row, pl.ds(kk * LANES, LANES)]
                            g[p][row, pl.ds(kk * LANES, LANES)] = blk * s16
                    return 0

                lax.fori_loop(0, t, t_body, 0)
                pltpu.async_copy(
                    g[p].reshape(t, dp, D),
                    out_hbm.at[pl.ds((b0 + i) * t, t)], wsem[p])
            return 0

        lax.fori_loop(0, bpw // 2, pair_body, 0)

        pltpu.make_async_copy(
            g[1].reshape(t, dp, D),
            out_hbm.at[pl.ds((b0 + bpw - 1) * t, t)], wsem[1]).wait()

    return sc_embed


def kernel(idx, num, table):
    b, t, dp = idx.shape
    n = b * t * dp
    idx_flat = idx.reshape(n).astype(jnp.int32)
    num_flat = num.reshape(n)
    out = _make_sc_kernel(b, t, dp, table.shape[0])(table, idx_flat, num_flat)
    return out.reshape(b, t, dp, D)


# speedup vs baseline: 2.1610x; 1.1469x over previous
import functools

import jax
import jax.numpy as jnp
from jax import lax
from jax.experimental import pallas as pl
from jax.experimental.pallas import tpu as pltpu
from jax.experimental.pallas import tpu_sc as plsc

D = 128
LANES = 16
COLB = D // LANES
NBUF = 4


def _make_sc_kernel(b, t, dp, vocab):
    info = plsc.get_sparse_core_info()
    nw = info.num_cores * info.num_subcores
    bpw = b // nw
    th = t // 2
    kh = th * dp
    nch = bpw * 2
    nfull = kh // LANES
    ktail = kh - nfull * LANES
    npw = bpw * t * dp
    assert nch % NBUF == 0

    mesh = plsc.VectorSubcoreMesh(core_axis_name="c", subcore_axis_name="s")

    @functools.partial(
        pl.kernel,
        mesh=mesh,
        out_type=jax.ShapeDtypeStruct((b * t, dp, D), jnp.float32),
        compiler_params=pltpu.CompilerParams(needs_layout_passes=False),
        scratch_types=[
            pltpu.VMEM((npw,), jnp.int32),
            pltpu.VMEM((npw,), jnp.float32),
            [pltpu.VMEM((kh,), jnp.int32)] * NBUF,
            [pltpu.VMEM((kh, D), jnp.float32)] * NBUF,
            pltpu.VMEM_SHARED((vocab, D), jnp.float32),
            [pltpu.SemaphoreType.DMA] * NBUF,
            [pltpu.SemaphoreType.DMA] * NBUF,
        ],
    )
    def sc_embed(table_hbm, idx_hbm, num_hbm, out_hbm,
                 idx_all, num_all, st, g, table_sp, gsem, wsem):
        wid = lax.axis_index("s") * info.num_cores + lax.axis_index("c")
        wbase = wid * npw
        bt0 = wid * bpw * t
        sid = lax.axis_index("s")
        vpt = vocab // info.num_subcores
        pltpu.sync_copy(table_hbm.at[pl.ds(sid * vpt, vpt)],
                        table_sp.at[pl.ds(sid * vpt, vpt)])
        pltpu.sync_copy(idx_hbm.at[pl.ds(wbase, npw)], idx_all)
        pltpu.sync_copy(num_hbm.at[pl.ds(wbase, npw)], num_all)
        plsc.subcore_barrier()
        lanes = lax.iota(jnp.int32, LANES)

        def stage(h, sp):
            cbase = h * kh
            for gg in range(nfull):
                pos = lanes + (cbase + gg * LANES)
                sp[pl.ds(gg * LANES, LANES)] = plsc.load_gather(idx_all, [pos])
            pos = jnp.minimum(lanes + (cbase + nfull * LANES), npw - 1)
            v = plsc.load_gather(idx_all, [pos])
            plsc.store_scatter(sp, [lanes + nfull * LANES], v, mask=lanes < ktail)

        def start_gather(h, r):
            stage(h, st[r])
            pltpu.async_copy(table_sp.at[st[r]], g[r], gsem[r])

        def drain_write(h, r):
            pltpu.make_async_copy(
                g[r].reshape(th, dp, D),
                out_hbm.at[pl.ds(bt0 + h * th, th)], wsem[r]).wait()

        start_gather(0, 0)
        start_gather(1, 1)

        def quad_body(s, _):
            for hh in range(NBUF):
                h = NBUF * s + hh
                r = hh
                r2 = (hh + 2) % NBUF

                if hh < 2:
                    @pl.when(s >= 1)
                    def _():
                        drain_write(h - 2, r2)
                    start_gather(h + 2, r2)
                else:
                    @pl.when(s < nch // NBUF - 1)
                    def _():
                        drain_write(h - 2, r2)
                        start_gather(h + 2, r2)

                pltpu.make_async_copy(table_sp.at[st[r]], g[r], gsem[r]).wait()

                def t_body(tt, _):
                    for d in range(dp):
                        row = tt * dp + d
                        s16 = plsc.load_gather(
                            num_all,
                            [jnp.full((LANES,), h * kh + row, jnp.int32)])
                        for kk in range(COLB):
                            blk = g[r][row, pl.ds(kk * LANES, LANES)]
                            g[r][row, pl.ds(kk * LANES, LANES)] = blk * s16
                    return 0

                lax.fori_loop(0, th, t_body, 0)
                pltpu.async_copy(
                    g[r].reshape(th, dp, D),
                    out_hbm.at[pl.ds(bt0 + h * th, th)], wsem[r])
            return 0

        lax.fori_loop(0, nch // NBUF, quad_body, 0)

        for j in range(NBUF):
            drain_write(nch - NBUF + j, j)

    return sc_embed


def kernel(idx, num, table):
    b, t, dp = idx.shape
    n = b * t * dp
    idx_flat = idx.reshape(n).astype(jnp.int32)
    num_flat = num.reshape(n)
    out = _make_sc_kernel(b, t, dp, table.shape[0])(table, idx_flat, num_flat)
    return out.reshape(b, t, dp, D)
